# Initial kernel scaffold; baseline (speedup 1.0000x reference)
import jax
import jax.numpy as jnp
from jax.experimental import pallas as pl


def kernel(x, codebooks, ema_cluster_size):
    raise NotImplementedError("write your pallas kernel here")


# speedup vs baseline: 1.1466x; 1.1466x over previous
import jax
import jax.numpy as jnp
from jax.experimental import pallas as pl

_B, _N, _F = 4, 1024, 256
_H, _M, _D = 8, 2048, 32
_T = 1024
_NT = (_B * _N) // _T


def _vqe_body(x_ref, c_ref, ema_ref, out_ref, idx_ref, loss_ref, perp_ref,
              repl_ref):
    h = pl.program_id(0)
    c = c_ref[0]
    l2_c = jnp.sum(c * c, axis=1)
    csum = jnp.sum(c, axis=0)

    counts = jnp.zeros((_M,), jnp.float32)
    loss_part = jnp.float32(0.0)
    for t in range(_NT):
        q = x_ref[0, pl.ds(t * _T, _T), :]
        dot = jax.lax.dot_general(q, c, (((1,), (1,)), ((), ())),
                                  preferred_element_type=jnp.float32)
        l2_q = jnp.sum(q * q, axis=1, keepdims=True)
        sim = -(l2_q + l2_c[None, :] - 2.0 * dot)
        iota = jax.lax.broadcasted_iota(jnp.int32, (_T, _M), 1)
        row_max = jnp.max(sim, axis=-1, keepdims=True)
        picked = jnp.where(sim == row_max, iota, _M)
        idxc = jnp.min(picked, axis=-1).astype(jnp.int32)
        idx_ref[0, 0, pl.ds(t * _T, _T)] = idxc
        cmp = (idxc[:, None] == iota).astype(jnp.float32)
        counts = counts + jnp.sum(cmp, axis=0)
        diff = q - csum[None, :]
        loss_part = loss_part + jnp.sum(diff * diff)
        out_ref[0, pl.ds(t * _T, _T), :] = jnp.broadcast_to(
            csum[None, :], (_T, _D))

    mean = counts * (1.0 / (_B * _N))
    ent = -jnp.sum(mean * jnp.log(mean + 1e-10))
    perp_ref[0, 0, :] = jnp.full((128,), jnp.exp(ent), jnp.float32)

    @pl.when(h == 0)
    def _():
        loss_ref[...] = jnp.zeros_like(loss_ref)

    loss_ref[...] += loss_part * (1.0 / (_B * _N * _F))

    expired = (ema_ref[0, 0, :] < 2.0).astype(jnp.int32)
    repl_ref[0, 0, :] = jnp.full((128,), jnp.sum(expired), jnp.int32)


def _run(xr, codebooks, ema3, interpret=False):
    return pl.pallas_call(
        _vqe_body,
        grid=(_H,),
        in_specs=[
            pl.BlockSpec((1, _B * _N, _D), lambda h: (h, 0, 0)),
            pl.BlockSpec((1, _M, _D), lambda h: (h, 0, 0)),
            pl.BlockSpec((1, 1, _M), lambda h: (h, 0, 0)),
        ],
        out_specs=[
            pl.BlockSpec((1, _B * _N, _D), lambda h: (h, 0, 0)),
            pl.BlockSpec((1, 1, _B * _N), lambda h: (h, 0, 0)),
            pl.BlockSpec((1, 1, 128), lambda h: (0, 0, 0)),
            pl.BlockSpec((1, 1, 128), lambda h: (h, 0, 0)),
            pl.BlockSpec((1, 1, 128), lambda h: (h, 0, 0)),
        ],
        out_shape=[
            jax.ShapeDtypeStruct((_H, _B * _N, _D), jnp.float32),
            jax.ShapeDtypeStruct((_H, 1, _B * _N), jnp.int32),
            jax.ShapeDtypeStruct((1, 1, 128), jnp.float32),
            jax.ShapeDtypeStruct((_H, 1, 128), jnp.float32),
            jax.ShapeDtypeStruct((_H, 1, 128), jnp.int32),
        ],
        interpret=interpret,
    )(xr, codebooks, ema3)


@jax.jit
def kernel(x, codebooks, ema_cluster_size):
    xr = x.reshape(_B, _N, _H, _D).transpose(2, 0, 1, 3).reshape(
        _H, _B * _N, _D)
    ema3 = ema_cluster_size.reshape(_H, 1, _M)
    out_r, idx_r, loss_acc, perp_r, repl_r = _run(xr, codebooks, ema3)
    out = out_r.reshape(_H, _B, _N, _D).transpose(1, 2, 0, 3).reshape(
        _B, _N, _F)
    indices = idx_r.reshape(_H, _B, _N).transpose(1, 0, 2)
    return out, indices, loss_acc[0, 0, 0], perp_r[:, 0, 0], repl_r[:, 0, 0]


# speedup vs baseline: 1.2857x; 1.1213x over previous
import jax
import jax.numpy as jnp
from jax.experimental import pallas as pl

_B, _N, _F = 4, 1024, 256
_H, _M, _D = 8, 2048, 32
_T = 1024
_NT = (_B * _N) // _T


def _vqe_body(x_ref, c_ref, ema_ref, out_ref, idx_ref, loss_ref, perp_ref,
              repl_ref):
    h = pl.program_id(0)
    c = c_ref[0]
    l2_c = jnp.sum(c * c, axis=1)
    csum = jnp.sum(c, axis=0)

    counts = jnp.zeros((_M,), jnp.float32)
    loss_part = jnp.float32(0.0)
    for t in range(_NT):
        q = x_ref[0, pl.ds(t * _T, _T), :]
        dot = jax.lax.dot_general(q, c, (((1,), (1,)), ((), ())),
                                  preferred_element_type=jnp.float32)
        l2_q = jnp.sum(q * q, axis=1, keepdims=True)
        sim = -(l2_q + l2_c[None, :] - 2.0 * dot)
        iota = jax.lax.broadcasted_iota(jnp.int32, (_T, _M), 1)
        idxc = jnp.argmax(sim, axis=-1).astype(jnp.int32)
        idx_ref[0, 0, pl.ds(t * _T, _T)] = idxc
        cmp = (idxc[:, None] == iota).astype(jnp.float32)
        counts = counts + jnp.sum(cmp, axis=0)
        diff = q - csum[None, :]
        loss_part = loss_part + jnp.sum(diff * diff)
        out_ref[0, pl.ds(t * _T, _T), :] = jnp.broadcast_to(
            csum[None, :], (_T, _D))

    mean = counts * (1.0 / (_B * _N))
    ent = -jnp.sum(mean * jnp.log(mean + 1e-10))
    perp_ref[0, 0, :] = jnp.full((128,), jnp.exp(ent), jnp.float32)

    @pl.when(h == 0)
    def _():
        loss_ref[...] = jnp.zeros_like(loss_ref)

    loss_ref[...] += loss_part * (1.0 / (_B * _N * _F))

    expired = (ema_ref[0, 0, :] < 2.0).astype(jnp.int32)
    repl_ref[0, 0, :] = jnp.full((128,), jnp.sum(expired), jnp.int32)


def _run(xr, codebooks, ema3, interpret=False):
    return pl.pallas_call(
        _vqe_body,
        grid=(_H,),
        in_specs=[
            pl.BlockSpec((1, _B * _N, _D), lambda h: (h, 0, 0)),
            pl.BlockSpec((1, _M, _D), lambda h: (h, 0, 0)),
            pl.BlockSpec((1, 1, _M), lambda h: (h, 0, 0)),
        ],
        out_specs=[
            pl.BlockSpec((1, _B * _N, _D), lambda h: (h, 0, 0)),
            pl.BlockSpec((1, 1, _B * _N), lambda h: (h, 0, 0)),
            pl.BlockSpec((1, 1, 128), lambda h: (0, 0, 0)),
            pl.BlockSpec((1, 1, 128), lambda h: (h, 0, 0)),
            pl.BlockSpec((1, 1, 128), lambda h: (h, 0, 0)),
        ],
        out_shape=[
            jax.ShapeDtypeStruct((_H, _B * _N, _D), jnp.float32),
            jax.ShapeDtypeStruct((_H, 1, _B * _N), jnp.int32),
            jax.ShapeDtypeStruct((1, 1, 128), jnp.float32),
            jax.ShapeDtypeStruct((_H, 1, 128), jnp.float32),
            jax.ShapeDtypeStruct((_H, 1, 128), jnp.int32),
        ],
        interpret=interpret,
    )(xr, codebooks, ema3)


@jax.jit
def kernel(x, codebooks, ema_cluster_size):
    xr = x.reshape(_B, _N, _H, _D).transpose(2, 0, 1, 3).reshape(
        _H, _B * _N, _D)
    ema3 = ema_cluster_size.reshape(_H, 1, _M)
    out_r, idx_r, loss_acc, perp_r, repl_r = _run(xr, codebooks, ema3)
    out = out_r.reshape(_H, _B, _N, _D).transpose(1, 2, 0, 3).reshape(
        _B, _N, _F)
    indices = idx_r.reshape(_H, _B, _N).transpose(1, 0, 2)
    return out, indices, loss_acc[0, 0, 0], perp_r[:, 0, 0], repl_r[:, 0, 0]


# speedup vs baseline: 1.3731x; 1.0679x over previous
import jax
import jax.numpy as jnp
from jax.experimental import pallas as pl

_B, _N, _F = 4, 1024, 256
_H, _M, _D = 8, 2048, 32


def _vqe_body(x_ref, c_ref, ema_ref, out_ref, idx_ref, loss_ref, perp_ref,
              repl_ref, counts_ref):
    b = pl.program_id(0)
    iota = jax.lax.broadcasted_iota(jnp.int32, (_N, _M), 1)
    ones_row = jnp.ones((1, _N), jnp.float32)

    @pl.when(b == 0)
    def _():
        counts_ref[...] = jnp.zeros_like(counts_ref)
        loss_ref[...] = jnp.zeros_like(loss_ref)
        expired = (ema_ref[...] < 2.0).astype(jnp.int32)
        repl_ref[...] = jnp.broadcast_to(
            jnp.sum(expired, axis=1, keepdims=True), (_H, 128))

    csums = []
    for h in range(_H):
        c = c_ref[h]
        l2_c = jnp.sum(c * c, axis=1)
        q = x_ref[0, :, h * _D:(h + 1) * _D]
        l2_q = jnp.sum(q * q, axis=1, keepdims=True)
        dot = jax.lax.dot_general(q, c, (((1,), (1,)), ((), ())),
                                  preferred_element_type=jnp.float32)
        sim = -(l2_q + l2_c[None, :] - 2.0 * dot)
        row_max = jnp.max(sim, axis=-1, keepdims=True)
        eqf = (sim == row_max).astype(jnp.float32)
        picked = jnp.where(sim == row_max, iota, _M)
        idx_ref[0, h, :] = jnp.min(picked, axis=-1).astype(jnp.int32)
        counts_ref[h, :] += jnp.dot(
            ones_row, eqf, preferred_element_type=jnp.float32)[0]
        csums.append(jnp.sum(c, axis=0))

    csum_flat = jnp.concatenate(csums)
    out_ref[0] = jnp.broadcast_to(csum_flat[None, :], (_N, _F))
    diff = x_ref[0] - csum_flat[None, :]
    loss_ref[...] += jnp.sum(diff * diff) * (1.0 / (_B * _N * _F))

    @pl.when(b == _B - 1)
    def _():
        mean = counts_ref[...] * (1.0 / (_B * _N))
        ent = -jnp.sum(mean * jnp.log(mean + 1e-10), axis=1, keepdims=True)
        perp_ref[...] = jnp.broadcast_to(jnp.exp(ent), (_H, 128))


def _run(x, codebooks, ema_cluster_size, interpret=False):
    from jax.experimental.pallas import tpu as pltpu
    return pl.pallas_call(
        _vqe_body,
        grid=(_B,),
        in_specs=[
            pl.BlockSpec((1, _N, _F), lambda b: (b, 0, 0)),
            pl.BlockSpec((_H, _M, _D), lambda b: (0, 0, 0)),
            pl.BlockSpec((_H, _M), lambda b: (0, 0)),
        ],
        out_specs=[
            pl.BlockSpec((1, _N, _F), lambda b: (b, 0, 0)),
            pl.BlockSpec((1, _H, _N), lambda b: (b, 0, 0)),
            pl.BlockSpec((1, 128), lambda b: (0, 0)),
            pl.BlockSpec((_H, 128), lambda b: (0, 0)),
            pl.BlockSpec((_H, 128), lambda b: (0, 0)),
        ],
        out_shape=[
            jax.ShapeDtypeStruct((_B, _N, _F), jnp.float32),
            jax.ShapeDtypeStruct((_B, _H, _N), jnp.int32),
            jax.ShapeDtypeStruct((1, 128), jnp.float32),
            jax.ShapeDtypeStruct((_H, 128), jnp.float32),
            jax.ShapeDtypeStruct((_H, 128), jnp.int32),
        ],
        scratch_shapes=[pltpu.VMEM((_H, _M), jnp.float32)],
        interpret=interpret,
    )(x, codebooks, ema_cluster_size)


@jax.jit
def kernel(x, codebooks, ema_cluster_size):
    out, idx, loss, perp, repl = _run(x, codebooks, ema_cluster_size)
    return out, idx, loss[0, 0], perp[:, 0], repl[:, 0]
